# baseline (speedup 1.0000x reference)
import functools

import jax
import jax.numpy as jnp
from jax.experimental import pallas as pl
from jax.experimental.pallas import tpu as pltpu

N_EMBD = 768
N_HEAD = 12
HEAD_SIZE = 64
N_EXPERTS = 3
F = 4 * N_EMBD
T = 2048

QT = 512
MT = 256
FC = 1536
NEG = -1e30


def _bdot(a, b, dims=None):
    a16 = a.astype(jnp.bfloat16)
    b16 = b.astype(jnp.bfloat16)
    if dims is None:
        return jax.lax.dot(a16, b16, preferred_element_type=jnp.float32)
    return jax.lax.dot_general(a16, b16, dims,
                               preferred_element_type=jnp.float32)

def _rms(x, w, eps=1e-6):
    return x * jax.lax.rsqrt(jnp.mean(x * x, axis=-1, keepdims=True) + eps) * w


def _qkv_kernel(x_ref, w_ref, wqkv_ref, o_ref):
    xn = _rms(x_ref[...], w_ref[...])
    o_ref[...] = _bdot(xn, wqkv_ref[...])


def _attn_kernel(q_ref, k_ref, v_ref, o_ref):
    qq = q_ref[...]
    kk = k_ref[...]
    vv = v_ref[...]
    outs = []
    for i in range(2):
        q = qq[:, i * HEAD_SIZE:(i + 1) * HEAD_SIZE]
        k = kk[:, i * HEAD_SIZE:(i + 1) * HEAD_SIZE]
        v = vv[:, i * HEAD_SIZE:(i + 1) * HEAD_SIZE]
        s = _bdot(q, k, (((1,), (1,)), ((), ()))) * 0.125
        m = jnp.max(s, axis=-1, keepdims=True)
        p = jnp.exp(s - m)
        l = jnp.sum(p, axis=-1, keepdims=True)
        o = _bdot(p, v)
        outs.append(o / l)
    o_ref[...] = jnp.concatenate(outs, axis=1)


def _proj_kernel(a_ref, wo_ref, bo_ref, x_ref, o_ref):
    o_ref[...] = (x_ref[...] + bo_ref[...] + _bdot(a_ref[...], wo_ref[...]))


def _gate_kernel(x_ref, w_ref, wg_ref, mask_ref, aux_ref):
    h2 = _rms(x_ref[...], w_ref[...])
    logits = jax.lax.dot(h2, wg_ref[...], preferred_element_type=jnp.float32)
    col = jax.lax.broadcasted_iota(jnp.int32, (1, 128), 1)
    logits = logits + jnp.where(col < N_EXPERTS, 0.0, NEG)
    m = jnp.max(logits, axis=-1, keepdims=True)
    e = jnp.exp(logits - m)
    probs = e / jnp.sum(e, axis=-1, keepdims=True)
    p0 = probs[:, 0:1]
    p1 = probs[:, 1:2]
    p2 = probs[:, 2:3]
    ex0 = (p1 > p0) & (p2 > p0)
    ex1 = (p0 >= p1) & (p2 > p1)
    ex2 = (p0 >= p2) & (p1 >= p2)
    pex = jnp.where(ex0, p0, jnp.where(ex1, p1, p2))
    denom = (p0 + p1 + p2) - pex
    m0 = jnp.where(ex0, 0.0, p0 / denom)
    m1 = jnp.where(ex1, 0.0, p1 / denom)
    m2 = jnp.where(ex2, 0.0, p2 / denom)
    mask_ref[...] = (jnp.where(col == 0, m0, 0.0) +
                     jnp.where(col == 1, m1, 0.0) +
                     jnp.where(col == 2, m2, 0.0))
    t0 = (p0 >= p1) & (p0 >= p2)
    t1 = jnp.logical_not(t0) & (p1 >= p2)
    t2 = jnp.logical_not(t0) & jnp.logical_not(t1)
    inv_t = 1.0 / T
    imp0 = jnp.sum(p0) * inv_t
    imp1 = jnp.sum(p1) * inv_t
    imp2 = jnp.sum(p2) * inv_t
    l0 = jnp.sum(t0.astype(jnp.float32)) * inv_t
    l1 = jnp.sum(t1.astype(jnp.float32)) * inv_t
    l2 = jnp.sum(t2.astype(jnp.float32)) * inv_t
    aux = N_EXPERTS * (imp0 * l0 + imp1 * l1 + imp2 * l2) * 0.01
    aux_ref[...] = jnp.full((1, 1), 1.0, jnp.float32) * aux


def _moe_kernel(x_ref, w_ref, mask_ref, w1_ref, b1_ref, w2_ref, b2_ref,
                o_ref, acc_ref):
    e = pl.program_id(0)
    fc = pl.program_id(1)
    t = pl.program_id(2)
    x = x_ref[...]
    h2 = _rms(x, w_ref[...])
    hid = _bdot(h2, w1_ref[0])
    hid = jnp.maximum(hid + b1_ref[0], 0.0)
    part = _bdot(hid, w2_ref[0])
    part = part + jnp.where(fc == 0, 1.0, 0.0) * b2_ref[0]
    col = jax.lax.broadcasted_iota(jnp.int32, (1, 128), 1)
    msel = jnp.sum(mask_ref[...] * (col == e).astype(jnp.float32),
                   axis=-1, keepdims=True)
    contrib = msel * part
    first = jnp.logical_and(e == 0, fc == 0)
    prev = jnp.where(first, x, acc_ref[pl.ds(t * MT, MT), :])
    new = prev + contrib
    acc_ref[pl.ds(t * MT, MT), :] = new
    o_ref[...] = new


def kernel(x, ln1_w, ln2_w, Wq, Wk, Wv, Wo, bo, Wg, W1, b1, W2, b2):
    x2 = x.reshape(T, N_EMBD)
    ln1 = ln1_w.reshape(1, N_EMBD)
    ln2 = ln2_w.reshape(1, N_EMBD)
    bo2 = bo.reshape(1, N_EMBD)
    wqkv = jnp.concatenate([
        jnp.transpose(Wq, (1, 0, 2)).reshape(N_EMBD, N_HEAD * HEAD_SIZE),
        jnp.transpose(Wk, (1, 0, 2)).reshape(N_EMBD, N_HEAD * HEAD_SIZE),
        jnp.transpose(Wv, (1, 0, 2)).reshape(N_EMBD, N_HEAD * HEAD_SIZE),
    ], axis=1)
    wg_pad = jnp.pad(Wg, ((0, 0), (0, 128 - N_EXPERTS)))

    qkv = pl.pallas_call(
        _qkv_kernel,
        grid=(T // QT,),
        in_specs=[
            pl.BlockSpec((QT, N_EMBD), lambda t: (t, 0)),
            pl.BlockSpec((1, N_EMBD), lambda t: (0, 0)),
            pl.BlockSpec((N_EMBD, 3 * N_EMBD), lambda t: (0, 0)),
        ],
        out_specs=pl.BlockSpec((QT, 3 * N_EMBD), lambda t: (t, 0)),
        out_shape=jax.ShapeDtypeStruct((T, 3 * N_EMBD), jnp.float32),
    )(x2, ln1, wqkv)

    attout = pl.pallas_call(
        _attn_kernel,
        grid=(N_HEAD // 2, T // QT),
        in_specs=[
            pl.BlockSpec((QT, 2 * HEAD_SIZE), lambda h, t: (t, h)),
            pl.BlockSpec((T, 2 * HEAD_SIZE), lambda h, t: (0, N_HEAD // 2 + h)),
            pl.BlockSpec((T, 2 * HEAD_SIZE), lambda h, t: (0, N_HEAD + h)),
        ],
        out_specs=pl.BlockSpec((QT, 2 * HEAD_SIZE), lambda h, t: (t, h)),
        out_shape=jax.ShapeDtypeStruct((T, N_EMBD), jnp.float32),
    )(qkv, qkv, qkv)

    x1 = pl.pallas_call(
        _proj_kernel,
        grid=(T // QT,),
        in_specs=[
            pl.BlockSpec((QT, N_EMBD), lambda t: (t, 0)),
            pl.BlockSpec((N_EMBD, N_EMBD), lambda t: (0, 0)),
            pl.BlockSpec((1, N_EMBD), lambda t: (0, 0)),
            pl.BlockSpec((QT, N_EMBD), lambda t: (t, 0)),
        ],
        out_specs=pl.BlockSpec((QT, N_EMBD), lambda t: (t, 0)),
        out_shape=jax.ShapeDtypeStruct((T, N_EMBD), jnp.float32),
    )(attout, Wo, bo2, x2)

    mask, aux = pl.pallas_call(
        _gate_kernel,
        in_specs=[
            pl.BlockSpec((T, N_EMBD), lambda: (0, 0)),
            pl.BlockSpec((1, N_EMBD), lambda: (0, 0)),
            pl.BlockSpec((N_EMBD, 128), lambda: (0, 0)),
        ],
        out_specs=[
            pl.BlockSpec((T, 128), lambda: (0, 0)),
            pl.BlockSpec((1, 1), lambda: (0, 0)),
        ],
        out_shape=[
            jax.ShapeDtypeStruct((T, 128), jnp.float32),
            jax.ShapeDtypeStruct((1, 1), jnp.float32),
        ],
    )(x1, ln2, wg_pad)

    out = pl.pallas_call(
        _moe_kernel,
        grid=(N_EXPERTS, F // FC, T // MT),
        in_specs=[
            pl.BlockSpec((MT, N_EMBD), lambda e, f, t: (t, 0)),
            pl.BlockSpec((1, N_EMBD), lambda e, f, t: (0, 0)),
            pl.BlockSpec((MT, 128), lambda e, f, t: (t, 0)),
            pl.BlockSpec((1, N_EMBD, FC), lambda e, f, t: (e, 0, f)),
            pl.BlockSpec((1, 1, FC), lambda e, f, t: (e, 0, f)),
            pl.BlockSpec((1, FC, N_EMBD), lambda e, f, t: (e, f, 0)),
            pl.BlockSpec((1, 1, N_EMBD), lambda e, f, t: (e, 0, 0)),
        ],
        out_specs=pl.BlockSpec((MT, N_EMBD), lambda e, f, t: (t, 0)),
        out_shape=jax.ShapeDtypeStruct((T, N_EMBD), jnp.float32),
        scratch_shapes=[pltpu.VMEM((T, N_EMBD), jnp.float32)],
    )(x1, ln2, mask, W1, b1.reshape(N_EXPERTS, 1, F), W2,
      b2.reshape(N_EXPERTS, 1, N_EMBD))

    return (out.reshape(1, T, N_EMBD), aux.reshape(()))


# speedup vs baseline: 1.1607x; 1.1607x over previous
import jax
import jax.numpy as jnp
from jax.experimental import pallas as pl
from jax.experimental.pallas import tpu as pltpu

N_EMBD = 768
N_HEAD = 12
HEAD_SIZE = 64
N_EXPERTS = 3
F = 4 * N_EMBD
T = 2048

QT = 512
MT = 256
FC = 1536
NF = F // FC
NEG = -1e30
NT_Q = T // QT
NT_M = T // MT


def _dot(a, b, dims=None):
    if dims is None:
        return jax.lax.dot(a, b, preferred_element_type=jnp.float32)
    return jax.lax.dot_general(a, b, dims,
                               preferred_element_type=jnp.float32)


def _rms(x, w, eps=1e-6):
    return x * jax.lax.rsqrt(jnp.mean(x * x, axis=-1, keepdims=True) + eps) * w


def _qkv_kernel(x_ref, w_ref, wqkv_ref, o_ref):
    xn = _rms(x_ref[...], w_ref[...])
    o_ref[...] = _dot(xn, wqkv_ref[...])


def _attn_gate_kernel(q_ref, k_ref, v_ref, x_ref, wo_ref, bo_ref,
                      ln2_ref, wg_ref, x1_ref, mask_ref, aux_ref,
                      stat_ref):
    t = pl.program_id(0)
    qq = q_ref[...]
    kk = k_ref[...]
    vv = v_ref[...]
    outs = []
    for h in range(N_HEAD):
        sl = slice(h * HEAD_SIZE, (h + 1) * HEAD_SIZE)
        q = qq[:, sl]
        k = kk[:, sl]
        v = vv[:, sl]
        s = _dot(q, k, (((1,), (1,)), ((), ()))) * 0.125
        m = jnp.max(s, axis=-1, keepdims=True)
        p = jnp.exp(s - m)
        l = jnp.sum(p, axis=-1, keepdims=True)
        outs.append(_dot(p, v) / l)
    att = jnp.concatenate(outs, axis=1)
    x1 = x_ref[...] + bo_ref[...] + _dot(att, wo_ref[...])
    x1_ref[...] = x1
    h2 = _rms(x1, ln2_ref[...])
    col = jax.lax.broadcasted_iota(jnp.int32, (1, 128), 1)
    logits = _dot(h2, wg_ref[...]) + jnp.where(col < N_EXPERTS, 0.0, NEG)
    m = jnp.max(logits, axis=-1, keepdims=True)
    e = jnp.exp(logits - m)
    probs = e / jnp.sum(e, axis=-1, keepdims=True)
    p0 = probs[:, 0:1]
    p1 = probs[:, 1:2]
    p2 = probs[:, 2:3]
    ex0 = (p1 > p0) & (p2 > p0)
    ex1 = (p0 >= p1) & (p2 > p1)
    ex2 = (p0 >= p2) & (p1 >= p2)
    pex = jnp.where(ex0, p0, jnp.where(ex1, p1, p2))
    denom = (p0 + p1 + p2) - pex
    m0 = jnp.where(ex0, 0.0, p0 / denom)
    m1 = jnp.where(ex1, 0.0, p1 / denom)
    m2 = jnp.where(ex2, 0.0, p2 / denom)
    mask_ref[...] = (jnp.where(col == 0, m0, 0.0) +
                     jnp.where(col == 1, m1, 0.0) +
                     jnp.where(col == 2, m2, 0.0))
    t0 = (p0 >= p1) & (p0 >= p2)
    t1 = jnp.logical_not(t0) & (p1 >= p2)
    t2 = jnp.logical_not(t0) & jnp.logical_not(t1)
    part = (jnp.where(col == 0, jnp.sum(p0), 0.0) +
            jnp.where(col == 1, jnp.sum(p1), 0.0) +
            jnp.where(col == 2, jnp.sum(p2), 0.0) +
            jnp.where(col == 3, jnp.sum(t0.astype(jnp.float32)), 0.0) +
            jnp.where(col == 4, jnp.sum(t1.astype(jnp.float32)), 0.0) +
            jnp.where(col == 5, jnp.sum(t2.astype(jnp.float32)), 0.0))
    prev = jnp.where(t == 0, jnp.zeros_like(part), stat_ref[...])
    stat = prev + part
    stat_ref[...] = stat

    @pl.when(t == NT_Q - 1)
    def _():
        imp = stat[:, 0:3]
        load = stat[:, 3:6]
        aux = (N_EXPERTS * 0.01 / (T * T)) * jnp.sum(imp * load)
        aux_ref[...] = jnp.full((1, 1), 1.0, jnp.float32) * aux


def _moe_kernel(x_ref, w_ref, mask_ref, w1_ref, b1_ref, w2_ref, b2_ref,
                o_ref, acc_ref, h2s_ref):
    e = pl.program_id(0)
    fc = pl.program_id(1)
    t = pl.program_id(2)
    first = jnp.logical_and(e == 0, fc == 0)
    last = jnp.logical_and(e == N_EXPERTS - 1, fc == NF - 1)
    rows = pl.ds(t * MT, MT)
    x = x_ref[...]
    h2_new = _rms(x, w_ref[...])
    h2 = jnp.where(first, h2_new, h2s_ref[rows, :])
    hid = jnp.maximum(_dot(h2, w1_ref[0]) + b1_ref[0], 0.0)
    part = _dot(hid, w2_ref[0])
    part = part + jnp.where(fc == 0, 1.0, 0.0) * b2_ref[0]
    col = jax.lax.broadcasted_iota(jnp.int32, (1, 128), 1)
    msel = jnp.sum(mask_ref[...] * (col == e).astype(jnp.float32),
                   axis=-1, keepdims=True)
    contrib = msel * part
    prev = jnp.where(first, x, acc_ref[rows, :])
    new = prev + contrib
    acc_ref[rows, :] = new

    @pl.when(first)
    def _():
        h2s_ref[rows, :] = h2_new

    @pl.when(last)
    def _():
        o_ref[...] = new


def kernel(x, ln1_w, ln2_w, Wq, Wk, Wv, Wo, bo, Wg, W1, b1, W2, b2):
    x2 = x.reshape(T, N_EMBD)
    ln1 = ln1_w.reshape(1, N_EMBD)
    ln2 = ln2_w.reshape(1, N_EMBD)
    bo2 = bo.reshape(1, N_EMBD)
    wqkv = jnp.concatenate([
        jnp.transpose(Wq, (1, 0, 2)).reshape(N_EMBD, N_HEAD * HEAD_SIZE),
        jnp.transpose(Wk, (1, 0, 2)).reshape(N_EMBD, N_HEAD * HEAD_SIZE),
        jnp.transpose(Wv, (1, 0, 2)).reshape(N_EMBD, N_HEAD * HEAD_SIZE),
    ], axis=1)
    wg_pad = jnp.pad(Wg, ((0, 0), (0, 128 - N_EXPERTS)))

    qkv = pl.pallas_call(
        _qkv_kernel,
        grid=(T // QT,),
        in_specs=[
            pl.BlockSpec((QT, N_EMBD), lambda t: (t, 0)),
            pl.BlockSpec((1, N_EMBD), lambda t: (0, 0)),
            pl.BlockSpec((N_EMBD, 3 * N_EMBD), lambda t: (0, 0)),
        ],
        out_specs=pl.BlockSpec((QT, 3 * N_EMBD), lambda t: (t, 0)),
        out_shape=jax.ShapeDtypeStruct((T, 3 * N_EMBD), jnp.float32),
    )(x2, ln1, wqkv)

    x1, mask, aux = pl.pallas_call(
        _attn_gate_kernel,
        grid=(NT_Q,),
        in_specs=[
            pl.BlockSpec((QT, N_EMBD), lambda t: (t, 0)),
            pl.BlockSpec((T, N_EMBD), lambda t: (0, 1)),
            pl.BlockSpec((T, N_EMBD), lambda t: (0, 2)),
            pl.BlockSpec((QT, N_EMBD), lambda t: (t, 0)),
            pl.BlockSpec((N_EMBD, N_EMBD), lambda t: (0, 0)),
            pl.BlockSpec((1, N_EMBD), lambda t: (0, 0)),
            pl.BlockSpec((1, N_EMBD), lambda t: (0, 0)),
            pl.BlockSpec((N_EMBD, 128), lambda t: (0, 0)),
        ],
        out_specs=[
            pl.BlockSpec((QT, N_EMBD), lambda t: (t, 0)),
            pl.BlockSpec((QT, 128), lambda t: (t, 0)),
            pl.BlockSpec((1, 1), lambda t: (0, 0)),
        ],
        out_shape=[
            jax.ShapeDtypeStruct((T, N_EMBD), jnp.float32),
            jax.ShapeDtypeStruct((T, 128), jnp.float32),
            jax.ShapeDtypeStruct((1, 1), jnp.float32),
        ],
        scratch_shapes=[pltpu.VMEM((1, 128), jnp.float32)],
    )(qkv, qkv, qkv, x2, Wo, bo2, ln2, wg_pad)

    out = pl.pallas_call(
        _moe_kernel,
        grid=(N_EXPERTS, NF, NT_M),
        in_specs=[
            pl.BlockSpec((MT, N_EMBD),
                         lambda e, f, t:
                         (jnp.where((e == 0) & (f == 0), t, 0), 0)),
            pl.BlockSpec((1, N_EMBD), lambda e, f, t: (0, 0)),
            pl.BlockSpec((MT, 128), lambda e, f, t: (t, 0)),
            pl.BlockSpec((1, N_EMBD, FC), lambda e, f, t: (e, 0, f)),
            pl.BlockSpec((1, 1, FC), lambda e, f, t: (e, 0, f)),
            pl.BlockSpec((1, FC, N_EMBD), lambda e, f, t: (e, f, 0)),
            pl.BlockSpec((1, 1, N_EMBD), lambda e, f, t: (e, 0, 0)),
        ],
        out_specs=pl.BlockSpec(
            (MT, N_EMBD),
            lambda e, f, t:
            (jnp.where((e == N_EXPERTS - 1) & (f == NF - 1), t, 0), 0)),
        out_shape=jax.ShapeDtypeStruct((T, N_EMBD), jnp.float32),
        scratch_shapes=[pltpu.VMEM((T, N_EMBD), jnp.float32),
                        pltpu.VMEM((T, N_EMBD), jnp.float32)],
    )(x1, ln2, mask, W1, b1.reshape(N_EXPERTS, 1, F), W2,
      b2.reshape(N_EXPERTS, 1, N_EMBD))

    return (out.reshape(1, T, N_EMBD), aux.reshape(()))
